# baseline (speedup 1.0000x reference)
import functools

import jax
import jax.numpy as jnp
from jax import lax
from jax.experimental import pallas as pl
from jax.experimental.pallas import tpu as pltpu
from jax.experimental.pallas import tpu_sc as plsc

_K = 1024
_LAMDA = 0.5
_T = 5.0


def _key_to_f32(key):
    bits = jnp.where(key >= jnp.uint32(0x80000000),
                     key ^ jnp.uint32(0x80000000), ~key)
    return jax.lax.bitcast_convert_type(bits, jnp.float32)


def _body(x_ref, xt_ref, out_ref, *, rows, v):
    scale = jnp.float32(1.0 / _T)
    kc = jnp.int32(_K)
    idx = jax.lax.broadcasted_iota(jnp.int32, (rows, v), 1)

    maxx = jnp.max(x_ref[...], axis=1, keepdims=True)
    maxt = jnp.max(xt_ref[...], axis=1, keepdims=True)
    sum_s = jnp.sum(jnp.exp((x_ref[...] - maxx) * scale), axis=1)

    def vbody(_, carry):
        lo, hi, cnt_lo = carry
        gap = hi - lo
        mid = lo + (gap >> 1) + (gap & jnp.uint32(1))
        fmid = _key_to_f32(mid)
        cnt = jnp.sum((xt_ref[...] >= fmid).astype(jnp.int32), axis=1,
                      keepdims=True)
        pred = cnt >= kc
        return (jnp.where(pred, mid, lo), jnp.where(pred, hi, mid - 1),
                jnp.where(pred, cnt, cnt_lo))

    tau, _, cnt_ge = jax.lax.fori_loop(
        0, 32, vbody,
        (jnp.zeros((rows, 1), jnp.uint32),
         jnp.full((rows, 1), 0xFFFFFFFF, jnp.uint32),
         jnp.full((rows, 1), v, jnp.int32)),
    )
    ftau = _key_to_f32(tau)

    def _no_tie(_):
        return jnp.full((rows, 1), v - 1, jnp.int32)

    def _tie(_):
        eq = xt_ref[...] == ftau
        cnt_eq = jnp.sum(eq.astype(jnp.int32), axis=1, keepdims=True)
        r = kc - (cnt_ge - cnt_eq)

        def ibody(_, carry):
            lo, hi = carry
            mid = (lo + hi) >> 1
            cnt = jnp.sum((eq & (idx <= mid)).astype(jnp.int32), axis=1,
                          keepdims=True)
            pred = cnt >= r
            return jnp.where(pred, lo, mid + 1), jnp.where(pred, mid, hi)

        istar, _ = jax.lax.fori_loop(
            0, 17, ibody,
            (jnp.zeros((rows, 1), jnp.int32),
             jnp.full((rows, 1), v - 1, jnp.int32)),
        )
        return istar

    istar = jax.lax.cond(jnp.all(cnt_ge == kc), _no_tie, _tie, 0)

    mask = (xt_ref[...] > ftau) | ((xt_ref[...] == ftau) & (idx <= istar))

    e_s = jnp.where(mask, jnp.exp((x_ref[...] - maxx) * scale), 0.0)
    a = jnp.sum(e_s, axis=1)
    b = jnp.sum(e_s * (x_ref[...] - xt_ref[...]), axis=1) * scale
    c = jnp.sum(jnp.where(mask, jnp.exp((xt_ref[...] - maxt) * scale), 0.0),
                axis=1)

    kl = b / a - jnp.log(a) + jnp.log(c) + (maxt[:, 0] - maxx[:, 0]) * scale
    ce_part = maxx[:, 0] * scale + jnp.log(sum_s)
    total = jnp.sum(ce_part + jnp.float32(_LAMDA * _T * _T) * kl)

    @pl.when(pl.program_id(0) == 0)
    def _init():
        out_ref[...] = jnp.zeros((1, 1), jnp.float32)

    out_ref[...] += total.reshape(1, 1)


def _sc_gather(xflat, flat_idx):
    bsz = flat_idx.shape[0]
    mesh = plsc.VectorSubcoreMesh(core_axis_name="c", subcore_axis_name="s")

    @functools.partial(
        pl.kernel, mesh=mesh,
        out_type=jax.ShapeDtypeStruct((bsz,), jnp.float32),
        scratch_types=[
            pltpu.VMEM((bsz,), jnp.int32),
            pltpu.VMEM((bsz,), jnp.float32),
            pltpu.SemaphoreType.DMA,
        ],
    )
    def gk(xflat_hbm, idx_hbm, out_hbm, idx_v, vals_v, sem):
        wid = lax.axis_index("s") * 2 + lax.axis_index("c")

        @pl.when(wid == 0)
        def _():
            pltpu.sync_copy(idx_hbm, idx_v)
            pltpu.async_copy(xflat_hbm.at[idx_v], vals_v, sem).wait()
            pltpu.sync_copy(vals_v, out_hbm)

    return gk(xflat, flat_idx)


def kernel(x, target, x_teacher):
    bsz, v = x.shape
    rows = 16
    grid = bsz // rows
    flat_idx = (jnp.arange(bsz, dtype=jnp.int32) * v
                + target.astype(jnp.int32))
    x_tgt = _sc_gather(x.reshape(-1), flat_idx)
    out = pl.pallas_call(
        functools.partial(_body, rows=rows, v=v),
        grid=(grid,),
        in_specs=[
            pl.BlockSpec((rows, v), lambda i: (i, 0)),
            pl.BlockSpec((rows, v), lambda i: (i, 0)),
        ],
        out_specs=pl.BlockSpec((1, 1), lambda i: (0, 0)),
        out_shape=jax.ShapeDtypeStruct((1, 1), jnp.float32),
        compiler_params=pltpu.CompilerParams(
            dimension_semantics=("arbitrary",)),
    )(x, x_teacher)
    scale = 1.0 / _T
    return (out[0, 0] - scale * jnp.sum(x_tgt)) / bsz


# speedup vs baseline: 1.1867x; 1.1867x over previous
import functools

import jax
import jax.numpy as jnp
from jax.experimental import pallas as pl
from jax.experimental.pallas import tpu as pltpu

_K = 1024
_LAMDA = 0.5
_T = 5.0


def _key_to_f32(key):
    bits = jnp.where(key >= jnp.uint32(0x80000000),
                     key ^ jnp.uint32(0x80000000), ~key)
    return jax.lax.bitcast_convert_type(bits, jnp.float32)


def _body(x_ref, xt_ref, tgt_ref, out_ref, *, rows, v):
    scale = jnp.float32(1.0 / _T)
    kc = jnp.int32(_K)
    idx = jax.lax.broadcasted_iota(jnp.int32, (rows, v), 1)

    maxx = jnp.max(x_ref[...], axis=1, keepdims=True)
    maxt = jnp.max(xt_ref[...], axis=1, keepdims=True)
    sum_s = jnp.sum(jnp.exp((x_ref[...] - maxx) * scale), axis=1)
    tgt = tgt_ref[...]
    x_tgt = jnp.sum(jnp.where(idx == tgt, x_ref[...], 0.0), axis=1)

    def vbody(_, carry):
        lo, hi, cnt_lo = carry
        gap = hi - lo
        mid = lo + (gap >> 1) + (gap & jnp.uint32(1))
        fmid = _key_to_f32(mid)
        cnt = jnp.sum((xt_ref[...] >= fmid).astype(jnp.int32), axis=1,
                      keepdims=True)
        pred = cnt >= kc
        return (jnp.where(pred, mid, lo), jnp.where(pred, hi, mid - 1),
                jnp.where(pred, cnt, cnt_lo))

    tau, _, cnt_ge = jax.lax.fori_loop(
        0, 32, vbody,
        (jnp.zeros((rows, 1), jnp.uint32),
         jnp.full((rows, 1), 0xFFFFFFFF, jnp.uint32),
         jnp.full((rows, 1), v, jnp.int32)),
    )
    ftau = _key_to_f32(tau)

    def _stats(mask):
        e_s = jnp.where(mask, jnp.exp((x_ref[...] - maxx) * scale), 0.0)
        a = jnp.sum(e_s, axis=1)
        b = jnp.sum(e_s * (x_ref[...] - xt_ref[...]), axis=1) * scale
        c = jnp.sum(
            jnp.where(mask, jnp.exp((xt_ref[...] - maxt) * scale), 0.0),
            axis=1)
        return a, b, c

    def _no_tie(_):
        return _stats(xt_ref[...] >= ftau)

    def _tie(_):
        eq = xt_ref[...] == ftau
        cnt_eq = jnp.sum(eq.astype(jnp.int32), axis=1, keepdims=True)
        r = kc - (cnt_ge - cnt_eq)

        def ibody(_, carry):
            lo, hi = carry
            mid = (lo + hi) >> 1
            cnt = jnp.sum((eq & (idx <= mid)).astype(jnp.int32), axis=1,
                          keepdims=True)
            pred = cnt >= r
            return jnp.where(pred, lo, mid + 1), jnp.where(pred, mid, hi)

        istar, _ = jax.lax.fori_loop(
            0, 17, ibody,
            (jnp.zeros((rows, 1), jnp.int32),
             jnp.full((rows, 1), v - 1, jnp.int32)),
        )
        return _stats((xt_ref[...] > ftau)
                      | ((xt_ref[...] == ftau) & (idx <= istar)))

    a, b, c = jax.lax.cond(jnp.all(cnt_ge == kc), _no_tie, _tie, 0)

    kl = b / a - jnp.log(a) + jnp.log(c) + (maxt[:, 0] - maxx[:, 0]) * scale
    ce = -((x_tgt - maxx[:, 0]) * scale - jnp.log(sum_s))
    total = jnp.sum(ce + jnp.float32(_LAMDA * _T * _T) * kl)

    @pl.when(pl.program_id(0) == 0)
    def _init():
        out_ref[...] = jnp.zeros((1, 1), jnp.float32)

    out_ref[...] += total.reshape(1, 1)


def kernel(x, target, x_teacher):
    bsz, v = x.shape
    rows = 16
    grid = bsz // rows
    tgt2 = target.reshape(bsz, 1).astype(jnp.int32)
    out = pl.pallas_call(
        functools.partial(_body, rows=rows, v=v),
        grid=(grid,),
        in_specs=[
            pl.BlockSpec((rows, v), lambda i: (i, 0)),
            pl.BlockSpec((rows, v), lambda i: (i, 0)),
            pl.BlockSpec((rows, 1), lambda i: (i, 0)),
        ],
        out_specs=pl.BlockSpec((1, 1), lambda i: (0, 0)),
        out_shape=jax.ShapeDtypeStruct((1, 1), jnp.float32),
        compiler_params=pltpu.CompilerParams(
            dimension_semantics=("arbitrary",)),
    )(x, x_teacher, tgt2)
    return out[0, 0] / bsz


# speedup vs baseline: 1.2164x; 1.0251x over previous
import functools

import jax
import jax.numpy as jnp
from jax.experimental import pallas as pl
from jax.experimental.pallas import tpu as pltpu

_K = 1024
_LAMDA = 0.5
_T = 5.0


def _key_to_f32(key):
    bits = jnp.where(key >= jnp.uint32(0x80000000),
                     key ^ jnp.uint32(0x80000000), ~key)
    return jax.lax.bitcast_convert_type(bits, jnp.float32)


def _body(x_ref, xt_ref, tgt_ref, out_ref, *, rows, v):
    scale = jnp.float32(1.0 / _T)
    kc = jnp.int32(_K)
    idx = jax.lax.broadcasted_iota(jnp.int32, (rows, v), 1)

    maxx = jnp.max(x_ref[...], axis=1, keepdims=True)
    maxt = jnp.max(xt_ref[...], axis=1, keepdims=True)
    sum_s = jnp.sum(jnp.exp((x_ref[...] - maxx) * scale), axis=1)
    tgt = tgt_ref[...]
    x_tgt = jnp.sum(jnp.where(idx == tgt, x_ref[...], 0.0), axis=1)

    def vbody(_, carry):
        lo, hi, cnt_lo = carry
        gap = hi - lo
        mid = lo + (gap >> 1) + (gap & jnp.uint32(1))
        fmid = _key_to_f32(mid)
        cnt = jnp.sum((xt_ref[...] >= fmid).astype(jnp.int32), axis=1,
                      keepdims=True)
        pred = cnt >= kc
        return (jnp.where(pred, mid, lo), jnp.where(pred, hi, mid - 1),
                jnp.where(pred, cnt, cnt_lo))

    tau, _, cnt_ge = jax.lax.fori_loop(
        0, 32, vbody,
        (jnp.zeros((rows, 1), jnp.uint32),
         jnp.full((rows, 1), 0xFFFFFFFF, jnp.uint32),
         jnp.full((rows, 1), v, jnp.int32)),
    )
    ftau = _key_to_f32(tau)

    def _no_tie(_):
        return jnp.full((rows, 1), v - 1, jnp.int32)

    def _tie(_):
        eq = xt_ref[...] == ftau
        cnt_eq = jnp.sum(eq.astype(jnp.int32), axis=1, keepdims=True)
        r = kc - (cnt_ge - cnt_eq)

        def ibody(_, carry):
            lo, hi = carry
            mid = (lo + hi) >> 1
            cnt = jnp.sum((eq & (idx <= mid)).astype(jnp.int32), axis=1,
                          keepdims=True)
            pred = cnt >= r
            return jnp.where(pred, lo, mid + 1), jnp.where(pred, mid, hi)

        istar, _ = jax.lax.fori_loop(
            0, 17, ibody,
            (jnp.zeros((rows, 1), jnp.int32),
             jnp.full((rows, 1), v - 1, jnp.int32)),
        )
        return istar

    istar = jax.lax.cond(jnp.all(cnt_ge == kc), _no_tie, _tie, 0)

    mask = (xt_ref[...] > ftau) | ((xt_ref[...] == ftau) & (idx <= istar))

    e_s = jnp.where(mask, jnp.exp((x_ref[...] - maxx) * scale), 0.0)
    a = jnp.sum(e_s, axis=1)
    b = jnp.sum(e_s * (x_ref[...] - xt_ref[...]), axis=1) * scale
    c = jnp.sum(jnp.where(mask, jnp.exp((xt_ref[...] - maxt) * scale), 0.0),
                axis=1)

    kl = b / a - jnp.log(a) + jnp.log(c) + (maxt[:, 0] - maxx[:, 0]) * scale
    ce = -((x_tgt - maxx[:, 0]) * scale - jnp.log(sum_s))
    total = jnp.sum(ce + jnp.float32(_LAMDA * _T * _T) * kl)

    @pl.when(pl.program_id(0) == 0)
    def _init():
        out_ref[...] = jnp.zeros((1, 1), jnp.float32)

    out_ref[...] += total.reshape(1, 1)


def kernel(x, target, x_teacher):
    bsz, v = x.shape
    rows = 16
    grid = bsz // rows
    tgt2 = target.reshape(bsz, 1).astype(jnp.int32)
    out = pl.pallas_call(
        functools.partial(_body, rows=rows, v=v),
        grid=(grid,),
        in_specs=[
            pl.BlockSpec((rows, v), lambda i: (i, 0)),
            pl.BlockSpec((rows, v), lambda i: (i, 0)),
            pl.BlockSpec((rows, 1), lambda i: (i, 0)),
        ],
        out_specs=pl.BlockSpec((1, 1), lambda i: (0, 0)),
        out_shape=jax.ShapeDtypeStruct((1, 1), jnp.float32),
        compiler_params=pltpu.CompilerParams(
            dimension_semantics=("arbitrary",)),
    )(x, x_teacher, tgt2)
    return out[0, 0] / bsz
